# speedup vs baseline: 4.1442x; 4.1442x over previous
import functools

import jax
import jax.numpy as jnp
from jax import lax
from jax.experimental import pallas as pl
from jax.experimental.pallas import tpu as pltpu
from jax.experimental.pallas import tpu_sc as plsc

_NFEAT = 128
_NATTRI = 1024
_NNODES = 4096
_E = 65536
_POS_W = 2.0
_PREC = lax.Precision.HIGHEST

_BN1 = 1024


def _k1_body(h_ref, ea_ref, en_ref, w0a_ref, w0b_ref, b0_ref,
             xw0_ref, loss_ref, acc_ref):
    j = pl.program_id(0)
    nj = pl.num_programs(0)
    h = h_ref[...]
    ea = ea_ref[...]
    en = en_ref[...]
    logits = lax.dot_general(ea, en, (((1,), (1,)), ((), ())),
                             preferred_element_type=jnp.float32,
                             precision=_PREC)
    sig = jax.nn.sigmoid(logits)
    mse = (h - sig) ** 2
    pos = jnp.sum(jnp.where(h == 1.0, mse, 0.0))
    neg = jnp.sum(jnp.where(h == 0.0, mse, 0.0))
    agg = lax.dot_general(h, ea, (((0,), (0,)), ((), ())),
                          preferred_element_type=jnp.float32,
                          precision=_PREC)
    xw0 = (lax.dot_general(en, w0a_ref[...], (((1,), (0,)), ((), ())),
                           preferred_element_type=jnp.float32,
                           precision=_PREC)
           + lax.dot_general(agg, w0b_ref[...], (((1,), (0,)), ((), ())),
                             preferred_element_type=jnp.float32,
                             precision=_PREC)
           + b0_ref[...])
    xw0_ref[...] = xw0

    @pl.when(j == 0)
    def _():
        acc_ref[0] = 0.0
        acc_ref[1] = 0.0

    acc_ref[0] += pos
    acc_ref[1] += neg

    @pl.when(j == nj - 1)
    def _():
        n = float(_NATTRI * _NNODES)
        val = _POS_W * acc_ref[0] / n + 0.1 * acc_ref[1] / n
        loss_ref[...] = jnp.full((1, 1), val, jnp.float32)


def _run_k1(H, ea, en, W0, b0):
    w0a = W0[:_NFEAT]
    w0b = W0[_NFEAT:]
    b0r = b0.reshape(1, _NFEAT)
    grid = _NNODES // _BN1
    xw0, loss = pl.pallas_call(
        _k1_body,
        grid=(grid,),
        in_specs=[
            pl.BlockSpec((_NATTRI, _BN1), lambda j: (0, j)),
            pl.BlockSpec((_NATTRI, _NFEAT), lambda j: (0, 0)),
            pl.BlockSpec((_BN1, _NFEAT), lambda j: (j, 0)),
            pl.BlockSpec((_NFEAT, _NFEAT), lambda j: (0, 0)),
            pl.BlockSpec((_NFEAT, _NFEAT), lambda j: (0, 0)),
            pl.BlockSpec((1, _NFEAT), lambda j: (0, 0)),
        ],
        out_specs=[
            pl.BlockSpec((_BN1, _NFEAT), lambda j: (j, 0)),
            pl.BlockSpec((1, 1), lambda j: (0, 0)),
        ],
        out_shape=[
            jax.ShapeDtypeStruct((_NNODES, _NFEAT), jnp.float32),
            jax.ShapeDtypeStruct((1, 1), jnp.float32),
        ],
        scratch_shapes=[pltpu.SMEM((2,), jnp.float32)],
    )(H, ea, en, w0a, w0b, b0r)
    return xw0, loss.reshape(())


_BM = 512


def _k2_body(g_ref, x_ref, w_ref, b_ref, e_ref, xw_ref):
    n = lax.dot_general(g_ref[...], x_ref[...], (((1,), (0,)), ((), ())),
                        preferred_element_type=jnp.float32, precision=_PREC)
    nrm = jnp.sqrt(jnp.sum(n * n, axis=1, keepdims=True))
    e_ref[...] = n / jnp.maximum(nrm, 1e-12)
    xw_ref[...] = (lax.dot_general(n, w_ref[...], (((1,), (0,)), ((), ())),
                                   preferred_element_type=jnp.float32,
                                   precision=_PREC)
                   + b_ref[...])


def _k3_body(g_ref, x_ref, e_ref):
    n = lax.dot_general(g_ref[...], x_ref[...], (((1,), (0,)), ((), ())),
                        preferred_element_type=jnp.float32, precision=_PREC)
    nrm = jnp.sqrt(jnp.sum(n * n, axis=1, keepdims=True))
    e_ref[...] = n / jnp.maximum(nrm, 1e-12)


def _run_layer0(G, xw0, W1, b1):
    grid = _NNODES // _BM
    e0, xw1 = pl.pallas_call(
        _k2_body,
        grid=(grid,),
        in_specs=[
            pl.BlockSpec((_BM, _NNODES), lambda i: (i, 0)),
            pl.BlockSpec((_NNODES, _NFEAT), lambda i: (0, 0)),
            pl.BlockSpec((_NFEAT, _NFEAT), lambda i: (0, 0)),
            pl.BlockSpec((1, _NFEAT), lambda i: (0, 0)),
        ],
        out_specs=[
            pl.BlockSpec((_BM, _NFEAT), lambda i: (i, 0)),
            pl.BlockSpec((_BM, _NFEAT), lambda i: (i, 0)),
        ],
        out_shape=[
            jax.ShapeDtypeStruct((_NNODES, _NFEAT), jnp.float32),
            jax.ShapeDtypeStruct((_NNODES, _NFEAT), jnp.float32),
        ],
    )(G, xw0, W1, b1.reshape(1, _NFEAT))
    return e0, xw1


def _run_layer1(G, xw1):
    grid = _NNODES // _BM
    return pl.pallas_call(
        _k3_body,
        grid=(grid,),
        in_specs=[
            pl.BlockSpec((_BM, _NNODES), lambda i: (i, 0)),
            pl.BlockSpec((_NNODES, _NFEAT), lambda i: (0, 0)),
        ],
        out_specs=pl.BlockSpec((_BM, _NFEAT), lambda i: (i, 0)),
        out_shape=jax.ShapeDtypeStruct((_NNODES, _NFEAT), jnp.float32),
    )(G, xw1)


_NC, _NS = 2, 16
_NW = _NC * _NS
_BPW = _E // _NW
_C = 128
_NCH = _BPW // _C


def _sc_gather8(e0, e1, ps, pd, ns, nd):
    idx2d = [a.reshape(_E // _C, _C) for a in (ps, pd, ns, nd)]
    mesh = plsc.VectorSubcoreMesh(core_axis_name="c", subcore_axis_name="s")
    out_t = [jax.ShapeDtypeStruct((_E, _NFEAT), jnp.float32)] * 8

    @functools.partial(
        pl.kernel, mesh=mesh, out_type=out_t,
        scratch_types=[
            pltpu.VMEM((_NCH, _C), jnp.int32),
            pltpu.VMEM((2, _C, _NFEAT), jnp.float32),
            pltpu.SemaphoreType.DMA,
            pltpu.SemaphoreType.DMA,
        ],
    )
    def gk(e0h, e1h, psh, pdh, nsh, ndh,
           o0, o1, o2, o3, o4, o5, o6, o7, idx_v, rows_v, gsem, osem):
        wid = lax.axis_index("s") * _NC + lax.axis_index("c")
        rowbase = wid * _NCH
        base = wid * _BPW
        pairs = [(psh, e0h, o0), (psh, e1h, o1), (pdh, e0h, o2), (pdh, e1h, o3),
                 (nsh, e0h, o4), (nsh, e1h, o5), (ndh, e0h, o6), (ndh, e1h, o7)]
        for p, (ih, th, oh) in enumerate(pairs):
            if p % 2 == 0:
                pltpu.sync_copy(ih.at[pl.ds(rowbase, _NCH)], idx_v)

            def body(c, carry, th=th, oh=oh):
                buf = lax.rem(c, 2)

                @pl.when(c >= 2)
                def _():
                    pltpu.make_async_copy(
                        rows_v.at[buf], oh.at[pl.ds(base, _C)], osem).wait()

                pltpu.async_copy(th.at[idx_v.at[c]], rows_v.at[buf],
                                 gsem).wait()
                pltpu.async_copy(rows_v.at[buf],
                                 oh.at[pl.ds(base + c * _C, _C)], osem)
                return carry

            lax.fori_loop(0, _NCH, body, 0)
            for _ in range(min(2, _NCH)):
                pltpu.make_async_copy(
                    rows_v.at[0], oh.at[pl.ds(base, _C)], osem).wait()

    return gk(e0, e1, *idx2d)


def kernel(hyper_node_G, pos_src, pos_dst, neg_src, neg_dst, Hyper_attr_H,
           emb_attri, emb_node, W0, b0, W1, b1):
    xw0, loss = _run_k1(Hyper_attr_H, emb_attri, emb_node, W0, b0)
    e0, xw1 = _run_layer0(hyper_node_G, xw0, W1, b1)
    e1 = _run_layer1(hyper_node_G, xw1)
    src0, src1, dst0, dst1, nsrc0, nsrc1, ndst0, ndst1 = _sc_gather8(
        e0, e1, pos_src, pos_dst, neg_src, neg_dst)
    return (src0, src1, dst0, dst1, nsrc0, nsrc1, ndst0, ndst1, loss)


# speedup vs baseline: 6.2391x; 1.5055x over previous
import functools

import jax
import jax.numpy as jnp
from jax import lax
from jax.experimental import pallas as pl
from jax.experimental.pallas import tpu as pltpu
from jax.experimental.pallas import tpu_sc as plsc

_NFEAT = 128
_NATTRI = 1024
_NNODES = 4096
_E = 65536
_POS_W = 2.0
_PREC = lax.Precision.DEFAULT

_BN1 = 1024


def _k1_body(h_ref, ea_ref, en_ref, w0a_ref, w0b_ref, b0_ref,
             xw0_ref, loss_ref, acc_ref):
    j = pl.program_id(0)
    nj = pl.num_programs(0)
    h = h_ref[...]
    ea = ea_ref[...]
    en = en_ref[...]
    logits = lax.dot_general(ea, en, (((1,), (1,)), ((), ())),
                             preferred_element_type=jnp.float32,
                             precision=_PREC)
    sig = jax.nn.sigmoid(logits)
    mse = (h - sig) ** 2
    pos = jnp.sum(jnp.where(h == 1.0, mse, 0.0))
    neg = jnp.sum(jnp.where(h == 0.0, mse, 0.0))
    agg = lax.dot_general(h, ea, (((0,), (0,)), ((), ())),
                          preferred_element_type=jnp.float32,
                          precision=_PREC)
    xw0 = (lax.dot_general(en, w0a_ref[...], (((1,), (0,)), ((), ())),
                           preferred_element_type=jnp.float32,
                           precision=_PREC)
           + lax.dot_general(agg, w0b_ref[...], (((1,), (0,)), ((), ())),
                             preferred_element_type=jnp.float32,
                             precision=_PREC)
           + b0_ref[...])
    xw0_ref[...] = xw0

    @pl.when(j == 0)
    def _():
        acc_ref[0] = 0.0
        acc_ref[1] = 0.0

    acc_ref[0] += pos
    acc_ref[1] += neg

    @pl.when(j == nj - 1)
    def _():
        n = float(_NATTRI * _NNODES)
        val = _POS_W * acc_ref[0] / n + 0.1 * acc_ref[1] / n
        loss_ref[...] = jnp.full((1, 1), val, jnp.float32)


def _run_k1(H, ea, en, W0, b0):
    w0a = W0[:_NFEAT]
    w0b = W0[_NFEAT:]
    b0r = b0.reshape(1, _NFEAT)
    grid = _NNODES // _BN1
    xw0, loss = pl.pallas_call(
        _k1_body,
        grid=(grid,),
        in_specs=[
            pl.BlockSpec((_NATTRI, _BN1), lambda j: (0, j)),
            pl.BlockSpec((_NATTRI, _NFEAT), lambda j: (0, 0)),
            pl.BlockSpec((_BN1, _NFEAT), lambda j: (j, 0)),
            pl.BlockSpec((_NFEAT, _NFEAT), lambda j: (0, 0)),
            pl.BlockSpec((_NFEAT, _NFEAT), lambda j: (0, 0)),
            pl.BlockSpec((1, _NFEAT), lambda j: (0, 0)),
        ],
        out_specs=[
            pl.BlockSpec((_BN1, _NFEAT), lambda j: (j, 0)),
            pl.BlockSpec((1, 1), lambda j: (0, 0)),
        ],
        out_shape=[
            jax.ShapeDtypeStruct((_NNODES, _NFEAT), jnp.float32),
            jax.ShapeDtypeStruct((1, 1), jnp.float32),
        ],
        scratch_shapes=[pltpu.SMEM((2,), jnp.float32)],
    )(H, ea, en, w0a, w0b, b0r)
    return xw0, loss.reshape(())


_BM = 512


def _k2_body(g_ref, x_ref, w_ref, b_ref, e_ref, xw_ref):
    n = lax.dot_general(g_ref[...], x_ref[...], (((1,), (0,)), ((), ())),
                        preferred_element_type=jnp.float32, precision=_PREC)
    nrm = jnp.sqrt(jnp.sum(n * n, axis=1, keepdims=True))
    e_ref[...] = n / jnp.maximum(nrm, 1e-12)
    xw_ref[...] = (lax.dot_general(n, w_ref[...], (((1,), (0,)), ((), ())),
                                   preferred_element_type=jnp.float32,
                                   precision=_PREC)
                   + b_ref[...])


def _k3_body(g_ref, x_ref, e_ref):
    n = lax.dot_general(g_ref[...], x_ref[...], (((1,), (0,)), ((), ())),
                        preferred_element_type=jnp.float32, precision=_PREC)
    nrm = jnp.sqrt(jnp.sum(n * n, axis=1, keepdims=True))
    e_ref[...] = n / jnp.maximum(nrm, 1e-12)


def _run_layer0(G, xw0, W1, b1):
    grid = _NNODES // _BM
    e0, xw1 = pl.pallas_call(
        _k2_body,
        grid=(grid,),
        in_specs=[
            pl.BlockSpec((_BM, _NNODES), lambda i: (i, 0)),
            pl.BlockSpec((_NNODES, _NFEAT), lambda i: (0, 0)),
            pl.BlockSpec((_NFEAT, _NFEAT), lambda i: (0, 0)),
            pl.BlockSpec((1, _NFEAT), lambda i: (0, 0)),
        ],
        out_specs=[
            pl.BlockSpec((_BM, _NFEAT), lambda i: (i, 0)),
            pl.BlockSpec((_BM, _NFEAT), lambda i: (i, 0)),
        ],
        out_shape=[
            jax.ShapeDtypeStruct((_NNODES, _NFEAT), jnp.float32),
            jax.ShapeDtypeStruct((_NNODES, _NFEAT), jnp.float32),
        ],
    )(G, xw0, W1, b1.reshape(1, _NFEAT))
    return e0, xw1


def _run_layer1(G, xw1):
    grid = _NNODES // _BM
    return pl.pallas_call(
        _k3_body,
        grid=(grid,),
        in_specs=[
            pl.BlockSpec((_BM, _NNODES), lambda i: (i, 0)),
            pl.BlockSpec((_NNODES, _NFEAT), lambda i: (0, 0)),
        ],
        out_specs=pl.BlockSpec((_BM, _NFEAT), lambda i: (i, 0)),
        out_shape=jax.ShapeDtypeStruct((_NNODES, _NFEAT), jnp.float32),
    )(G, xw1)


_NC, _NS = 2, 16
_NW = _NC * _NS
_BPW = _E // _NW
_C = 128
_NCH = _BPW // _C


def _sc_gather8(e0, e1, ps, pd, ns, nd):
    idx2d = [a.reshape(_E // _C, _C) for a in (ps, pd, ns, nd)]
    mesh = plsc.VectorSubcoreMesh(core_axis_name="c", subcore_axis_name="s")
    out_t = [jax.ShapeDtypeStruct((_E, _NFEAT), jnp.float32)] * 8

    @functools.partial(
        pl.kernel, mesh=mesh, out_type=out_t,
        scratch_types=[
            pltpu.VMEM((_NCH, _C), jnp.int32),
            pltpu.VMEM((4, _C, _NFEAT), jnp.float32),
            pltpu.SemaphoreType.DMA,
            pltpu.SemaphoreType.DMA,
        ],
    )
    def gk(e0h, e1h, psh, pdh, nsh, ndh,
           o0, o1, o2, o3, o4, o5, o6, o7, idx_v, rows_v, gsem, osem):
        wid = lax.axis_index("s") * _NC + lax.axis_index("c")
        rowbase = wid * _NCH
        base = wid * _BPW
        pairs = [(psh, e0h, o0), (psh, e1h, o1), (pdh, e0h, o2), (pdh, e1h, o3),
                 (nsh, e0h, o4), (nsh, e1h, o5), (ndh, e0h, o6), (ndh, e1h, o7)]
        for p, (ih, th, oh) in enumerate(pairs):
            if p % 2 == 0:
                pltpu.sync_copy(ih.at[pl.ds(rowbase, _NCH)], idx_v)

            pltpu.async_copy(th.at[idx_v.at[0]], rows_v.at[0], gsem)
            pltpu.async_copy(th.at[idx_v.at[1]], rows_v.at[1], gsem)

            def body(c, carry, th=th, oh=oh):
                buf = lax.rem(c, 4)
                pltpu.make_async_copy(th.at[idx_v.at[c]], rows_v.at[buf],
                                      gsem).wait()
                pltpu.async_copy(rows_v.at[buf],
                                 oh.at[pl.ds(base + c * _C, _C)], osem)

                @pl.when(c + 2 < _NCH)
                def _():
                    nbuf = lax.rem(c + 2, 4)

                    @pl.when(c >= 2)
                    def _():
                        pltpu.make_async_copy(
                            rows_v.at[nbuf], oh.at[pl.ds(base, _C)],
                            osem).wait()

                    pltpu.async_copy(th.at[idx_v.at[c + 2]], rows_v.at[nbuf],
                                     gsem)

                return carry

            lax.fori_loop(0, _NCH, body, 0)
            for _ in range(min(4, _NCH)):
                pltpu.make_async_copy(
                    rows_v.at[0], oh.at[pl.ds(base, _C)], osem).wait()

    return gk(e0, e1, *idx2d)


def kernel(hyper_node_G, pos_src, pos_dst, neg_src, neg_dst, Hyper_attr_H,
           emb_attri, emb_node, W0, b0, W1, b1):
    xw0, loss = _run_k1(Hyper_attr_H, emb_attri, emb_node, W0, b0)
    e0, xw1 = _run_layer0(hyper_node_G, xw0, W1, b1)
    e1 = _run_layer1(hyper_node_G, xw1)
    src0, src1, dst0, dst1, nsrc0, nsrc1, ndst0, ndst1 = _sc_gather8(
        e0, e1, pos_src, pos_dst, neg_src, neg_dst)
    return (src0, src1, dst0, dst1, nsrc0, nsrc1, ndst0, ndst1, loss)


# speedup vs baseline: 8.8328x; 1.4157x over previous
import functools

import jax
import jax.numpy as jnp
from jax import lax
from jax.experimental import pallas as pl
from jax.experimental.pallas import tpu as pltpu
from jax.experimental.pallas import tpu_sc as plsc

_NFEAT = 128
_NATTRI = 1024
_NNODES = 4096
_E = 65536
_POS_W = 2.0
_PREC = lax.Precision.DEFAULT

_BN1 = 1024


def _k1_body(h_ref, ea_ref, en_ref, w0a_ref, w0b_ref, b0_ref,
             xw0_ref, loss_ref, acc_ref):
    j = pl.program_id(0)
    nj = pl.num_programs(0)
    h = h_ref[...]
    ea = ea_ref[...]
    en = en_ref[...]
    logits = lax.dot_general(ea, en, (((1,), (1,)), ((), ())),
                             preferred_element_type=jnp.float32,
                             precision=_PREC)
    sig = jax.nn.sigmoid(logits)
    mse = (h - sig) ** 2
    pos = jnp.sum(jnp.where(h == 1.0, mse, 0.0))
    neg = jnp.sum(jnp.where(h == 0.0, mse, 0.0))
    agg = lax.dot_general(h, ea, (((0,), (0,)), ((), ())),
                          preferred_element_type=jnp.float32,
                          precision=_PREC)
    xw0 = (lax.dot_general(en, w0a_ref[...], (((1,), (0,)), ((), ())),
                           preferred_element_type=jnp.float32,
                           precision=_PREC)
           + lax.dot_general(agg, w0b_ref[...], (((1,), (0,)), ((), ())),
                             preferred_element_type=jnp.float32,
                             precision=_PREC)
           + b0_ref[...])
    xw0_ref[...] = xw0

    @pl.when(j == 0)
    def _():
        acc_ref[0] = 0.0
        acc_ref[1] = 0.0

    acc_ref[0] += pos
    acc_ref[1] += neg

    @pl.when(j == nj - 1)
    def _():
        n = float(_NATTRI * _NNODES)
        val = _POS_W * acc_ref[0] / n + 0.1 * acc_ref[1] / n
        loss_ref[...] = jnp.full((1, 1), val, jnp.float32)


def _run_k1(H, ea, en, W0, b0):
    w0a = W0[:_NFEAT]
    w0b = W0[_NFEAT:]
    b0r = b0.reshape(1, _NFEAT)
    grid = _NNODES // _BN1
    xw0, loss = pl.pallas_call(
        _k1_body,
        grid=(grid,),
        in_specs=[
            pl.BlockSpec((_NATTRI, _BN1), lambda j: (0, j)),
            pl.BlockSpec((_NATTRI, _NFEAT), lambda j: (0, 0)),
            pl.BlockSpec((_BN1, _NFEAT), lambda j: (j, 0)),
            pl.BlockSpec((_NFEAT, _NFEAT), lambda j: (0, 0)),
            pl.BlockSpec((_NFEAT, _NFEAT), lambda j: (0, 0)),
            pl.BlockSpec((1, _NFEAT), lambda j: (0, 0)),
        ],
        out_specs=[
            pl.BlockSpec((_BN1, _NFEAT), lambda j: (j, 0)),
            pl.BlockSpec((1, 1), lambda j: (0, 0)),
        ],
        out_shape=[
            jax.ShapeDtypeStruct((_NNODES, _NFEAT), jnp.float32),
            jax.ShapeDtypeStruct((1, 1), jnp.float32),
        ],
        scratch_shapes=[pltpu.SMEM((2,), jnp.float32)],
    )(H, ea, en, w0a, w0b, b0r)
    return xw0, loss.reshape(())


_BM = 512


def _k2_body(g_ref, x_ref, w_ref, b_ref, e_ref, xw_ref):
    n = lax.dot_general(g_ref[...], x_ref[...], (((1,), (0,)), ((), ())),
                        preferred_element_type=jnp.float32, precision=_PREC)
    nrm = jnp.sqrt(jnp.sum(n * n, axis=1, keepdims=True))
    e_ref[...] = n / jnp.maximum(nrm, 1e-12)
    xw_ref[...] = (lax.dot_general(n, w_ref[...], (((1,), (0,)), ((), ())),
                                   preferred_element_type=jnp.float32,
                                   precision=_PREC)
                   + b_ref[...])


def _k3_body(g_ref, x_ref, e_ref):
    n = lax.dot_general(g_ref[...], x_ref[...], (((1,), (0,)), ((), ())),
                        preferred_element_type=jnp.float32, precision=_PREC)
    nrm = jnp.sqrt(jnp.sum(n * n, axis=1, keepdims=True))
    e_ref[...] = n / jnp.maximum(nrm, 1e-12)


def _run_layer0(G, xw0, W1, b1):
    grid = _NNODES // _BM
    e0, xw1 = pl.pallas_call(
        _k2_body,
        grid=(grid,),
        in_specs=[
            pl.BlockSpec((_BM, _NNODES), lambda i: (i, 0)),
            pl.BlockSpec((_NNODES, _NFEAT), lambda i: (0, 0)),
            pl.BlockSpec((_NFEAT, _NFEAT), lambda i: (0, 0)),
            pl.BlockSpec((1, _NFEAT), lambda i: (0, 0)),
        ],
        out_specs=[
            pl.BlockSpec((_BM, _NFEAT), lambda i: (i, 0)),
            pl.BlockSpec((_BM, _NFEAT), lambda i: (i, 0)),
        ],
        out_shape=[
            jax.ShapeDtypeStruct((_NNODES, _NFEAT), jnp.float32),
            jax.ShapeDtypeStruct((_NNODES, _NFEAT), jnp.float32),
        ],
    )(G, xw0, W1, b1.reshape(1, _NFEAT))
    return e0, xw1


def _run_layer1(G, xw1):
    grid = _NNODES // _BM
    return pl.pallas_call(
        _k3_body,
        grid=(grid,),
        in_specs=[
            pl.BlockSpec((_BM, _NNODES), lambda i: (i, 0)),
            pl.BlockSpec((_NNODES, _NFEAT), lambda i: (0, 0)),
        ],
        out_specs=pl.BlockSpec((_BM, _NFEAT), lambda i: (i, 0)),
        out_shape=jax.ShapeDtypeStruct((_NNODES, _NFEAT), jnp.float32),
    )(G, xw1)


_NC, _NS = 2, 16
_NW = _NC * _NS
_BPW = _E // _NW
_C = 128
_NCH = _BPW // _C


def _sc_gather4(tbl, ps2d, pd2d, ns2d, nd2d):
    mesh = plsc.VectorSubcoreMesh(core_axis_name="c", subcore_axis_name="s")
    out_t = [jax.ShapeDtypeStruct((_E, _NFEAT), jnp.float32)] * 4

    @functools.partial(
        pl.kernel, mesh=mesh, out_type=out_t,
        scratch_types=[
            pltpu.VMEM((_NCH, _C), jnp.int32),
            pltpu.VMEM((4, _C, _NFEAT), jnp.float32),
            pltpu.VMEM_SHARED((_NNODES, _NFEAT), jnp.float32),
            pltpu.SemaphoreType.DMA,
            pltpu.SemaphoreType.DMA,
        ],
    )
    def gk(tblh, i0h, i1h, i2h, i3h,
           o0, o1, o2, o3, idx_v, rows_v, tbl_s, gsem, osem):
        sid = lax.axis_index("s")

        @pl.when(sid == 0)
        def _():
            pltpu.sync_copy(tblh, tbl_s)

        plsc.subcore_barrier()
        wid = sid * _NC + lax.axis_index("c")
        rowbase = wid * _NCH
        base = wid * _BPW
        for ih, oh in ((i0h, o0), (i1h, o1), (i2h, o2), (i3h, o3)):
            pltpu.sync_copy(ih.at[pl.ds(rowbase, _NCH)], idx_v)

            pltpu.async_copy(tbl_s.at[idx_v.at[0]], rows_v.at[0], gsem)
            pltpu.async_copy(tbl_s.at[idx_v.at[1]], rows_v.at[1], gsem)

            def body(c, carry, oh=oh):
                buf = lax.rem(c, 4)
                pltpu.make_async_copy(tbl_s.at[idx_v.at[c]], rows_v.at[buf],
                                      gsem).wait()
                pltpu.async_copy(rows_v.at[buf],
                                 oh.at[pl.ds(base + c * _C, _C)], osem)

                @pl.when(c + 2 < _NCH)
                def _():
                    nbuf = lax.rem(c + 2, 4)

                    @pl.when(c >= 2)
                    def _():
                        pltpu.make_async_copy(
                            rows_v.at[nbuf], oh.at[pl.ds(base, _C)],
                            osem).wait()

                    pltpu.async_copy(tbl_s.at[idx_v.at[c + 2]],
                                     rows_v.at[nbuf], gsem)

                return carry

            lax.fori_loop(0, _NCH, body, 0)
            for _ in range(min(4, _NCH)):
                pltpu.make_async_copy(
                    rows_v.at[0], oh.at[pl.ds(base, _C)], osem).wait()

    return gk(tbl, ps2d, pd2d, ns2d, nd2d)


def kernel(hyper_node_G, pos_src, pos_dst, neg_src, neg_dst, Hyper_attr_H,
           emb_attri, emb_node, W0, b0, W1, b1):
    idx2d = [a.reshape(_E // _C, _C)
             for a in (pos_src, pos_dst, neg_src, neg_dst)]
    xw0, loss = _run_k1(Hyper_attr_H, emb_attri, emb_node, W0, b0)
    e0, xw1 = _run_layer0(hyper_node_G, xw0, W1, b1)
    src0, dst0, nsrc0, ndst0 = _sc_gather4(e0, *idx2d)
    e1 = _run_layer1(hyper_node_G, xw1)
    src1, dst1, nsrc1, ndst1 = _sc_gather4(e1, *idx2d)
    return (src0, src1, dst0, dst1, nsrc0, nsrc1, ndst0, ndst1, loss)
